# baseline (speedup 1.0000x reference)
import functools

import jax
import jax.numpy as jnp
from jax import lax
from jax.experimental import pallas as pl
from jax.experimental.pallas import tpu as pltpu
from jax.experimental.pallas import tpu_sc as plsc

N = 16384
NM = 6
NP = 30
NC = 2
NS = 16
L = 16
NW = NC * NS
RPW = N // NW
C = 64
NCHUNK = RPW // C
GRP = C // L

CLS_TH = 2.0
CLS_IGNORE = 0.2
MGN = 0.2
KP_STEPS = (NP - 1, 9, 19, NP - 1)


def _sqrtv(x):
    i = plsc.bitcast(x, jnp.int32)
    i = 0x5F3759DF - lax.shift_right_logical(i, 1)
    y = plsc.bitcast(i, jnp.float32)
    for _ in range(3):
        y = y * (1.5 - 0.5 * x * y * y)
    return jnp.where(x > 0.0, x * y, 0.0)


def _sl1(d):
    ad = jnp.abs(d)
    return jnp.where(ad < 1.0, 0.5 * d * d, ad - 0.5)


def _body(cls_h, reg_h, kp_h, gt_h, hp_h, um_h, out_h,
          reg_v, gt_v, hp_v, cls_v, kp_v, um_v, outv):
    cid = lax.axis_index("c")
    sid = lax.axis_index("s")
    wid = sid * NC + cid
    base = wid * RPW

    lane = jnp.arange(L, dtype=jnp.int32)
    zf = jnp.zeros((L,), jnp.float32)

    def chunk_step(ck, accs):
        a_cls, a_reg, a_kp, a_ncls, a_nreg, a_nkp = accs
        r0 = base + ck * C
        pltpu.sync_copy(reg_h.at[pl.ds(r0 * (NM * NP * 2), C * NM * NP * 2)], reg_v)
        pltpu.sync_copy(gt_h.at[pl.ds(r0 * (NP * 2), C * NP * 2)], gt_v)
        pltpu.sync_copy(hp_h.at[pl.ds(r0 * NP, C * NP)], hp_v)
        pltpu.sync_copy(cls_h.at[pl.ds(r0 * NM, C * NM)], cls_v)
        pltpu.sync_copy(kp_h.at[pl.ds(r0 * (NM * 4 * 2), C * NM * 4 * 2)], kp_v)
        pltpu.sync_copy(um_h.at[pl.ds(r0, C)], um_v)

        for g in range(GRP):
            r = lane + g * L
            r30 = r * NP
            r360 = r * (NM * NP * 2)

            li = jnp.full((L,), NP - 1, jnp.int32)
            anyb = jnp.zeros((L,), jnp.int32) > 0
            for t in range(NP):
                hb = plsc.load_gather(hp_v, [r30 + t]) == 1
                li = jnp.where(hb, t, li)
                anyb = anyb | hb
            umb = plsc.load_gather(um_v, [r]) != 0
            valid = umb & anyb

            gtb = (r30 + li) * 2
            gx = plsc.load_gather(gt_v, [gtb])
            gy = plsc.load_gather(gt_v, [gtb + 1])
            rb = r360 + li * 2
            d2s = []
            for j in range(NM):
                rx = plsc.load_gather(reg_v, [rb + j * (NP * 2)])
                ry = plsc.load_gather(reg_v, [rb + j * (NP * 2) + 1])
                dx = rx - gx
                dy = ry - gy
                d2s.append(dx * dx + dy * dy)
            mind2 = d2s[0]
            minid = jnp.zeros((L,), jnp.int32)
            for j in range(1, NM):
                lt = d2s[j] < mind2
                mind2 = jnp.where(lt, d2s[j], mind2)
                minid = jnp.where(lt, j, minid)
            mind = _sqrtv(mind2)

            mask0 = (mind < CLS_TH) & valid
            cmin = plsc.load_gather(cls_v, [r * NM + minid])
            for j in range(NM):
                cj = plsc.load_gather(cls_v, [r * NM + j])
                mgn = cmin - cj
                dj = _sqrtv(d2s[j])
                mm = mask0 & ((dj - mind) > CLS_IGNORE) & (mgn < MGN)
                a_cls = a_cls + jnp.where(mm, MGN - mgn, zf)
                a_ncls = a_ncls + jnp.where(mm, 1.0, zf)

            rb2 = r360 + minid * (NP * 2)
            for t in range(NP):
                rx = plsc.load_gather(reg_v, [rb2 + 2 * t])
                ry = plsc.load_gather(reg_v, [rb2 + 2 * t + 1])
                gx2 = plsc.load_gather(gt_v, [r30 * 2 + 2 * t])
                gy2 = plsc.load_gather(gt_v, [r30 * 2 + 2 * t + 1])
                mk = (plsc.load_gather(hp_v, [r30 + t]) == 1) & valid
                e = _sl1(rx - gx2) + _sl1(ry - gy2)
                a_reg = a_reg + jnp.where(mk, e, zf)
                a_nreg = a_nreg + jnp.where(mk, 1.0, zf)

            kb = r * (NM * 4 * 2) + minid * 8
            for s, ts in enumerate(KP_STEPS):
                kx = plsc.load_gather(kp_v, [kb + 2 * s])
                ky = plsc.load_gather(kp_v, [kb + 2 * s + 1])
                gxs = plsc.load_gather(gt_v, [(r30 + ts) * 2])
                gys = plsc.load_gather(gt_v, [(r30 + ts) * 2 + 1])
                mk = (plsc.load_gather(hp_v, [r30 + ts]) == 1) & valid
                e = _sl1(kx - gxs) + _sl1(ky - gys)
                a_kp = a_kp + jnp.where(mk, e, zf)
                a_nkp = a_nkp + jnp.where(mk, 1.0, zf)

        return (a_cls, a_reg, a_kp, a_ncls, a_nreg, a_nkp)

    accs = lax.fori_loop(0, NCHUNK, chunk_step,
                         (zf, zf, zf, zf, zf, zf))
    for i in range(6):
        outv[i, :] = accs[i]
    pltpu.sync_copy(outv, out_h.at[wid])


@jax.jit
def _run(cls_f, reg_f, kp_f, gt_f, hp_f, um_f):
    mesh = plsc.VectorSubcoreMesh(core_axis_name="c", subcore_axis_name="s",
                                  num_cores=NC, num_subcores=NS)
    f = pl.kernel(
        _body,
        out_type=jax.ShapeDtypeStruct((NW, 6, L), jnp.float32),
        mesh=mesh,
        scratch_types=[
            pltpu.VMEM((C * NM * NP * 2,), jnp.float32),
            pltpu.VMEM((C * NP * 2,), jnp.float32),
            pltpu.VMEM((C * NP,), jnp.int32),
            pltpu.VMEM((C * NM,), jnp.float32),
            pltpu.VMEM((C * NM * 4 * 2,), jnp.float32),
            pltpu.VMEM((C,), jnp.int32),
            pltpu.VMEM((6, L), jnp.float32),
        ],
        compiler_params=pltpu.CompilerParams(needs_layout_passes=False),
    )
    return f(cls_f, reg_f, kp_f, gt_f, hp_f, um_f)


def kernel(cls, reg, key_points, gt_preds, has_preds, update_mask):
    out = _run(
        cls.reshape(-1),
        reg.reshape(-1),
        key_points.reshape(-1),
        gt_preds.reshape(-1),
        has_preds.reshape(-1).astype(jnp.int32),
        update_mask.reshape(-1).astype(jnp.int32),
    )
    s = out.sum(axis=(0, 2))
    return (s[0], s[1], s[2],
            s[3].astype(jnp.int32), s[4].astype(jnp.int32),
            s[5].astype(jnp.int32))


# speedup vs baseline: 53.7938x; 53.7938x over previous
import jax
import jax.numpy as jnp
from jax import lax
from jax.experimental import pallas as pl
from jax.experimental.pallas import tpu as pltpu
from jax.experimental.pallas import tpu_sc as plsc

N = 16384
NM = 6
NP = 30
NC = 2
NS = 16
L = 16
NW = NC * NS
B = 128
NB = N // B
BPW = NB // NW
GRP = B // L

CLS_TH = 2.0
CLS_IGNORE = 0.2
MGN = 0.2
KP_STEPS = (NP - 1, 9, 19, NP - 1)


def _sqrtv(x):
    i = plsc.bitcast(x, jnp.int32)
    i = 0x5F3759DF - lax.shift_right_logical(i, 1)
    y = plsc.bitcast(i, jnp.float32)
    for _ in range(3):
        y = y * (1.5 - 0.5 * x * y * y)
    return jnp.where(x > 0.0, x * y, 0.0)


def _sl1(d):
    ad = jnp.abs(d)
    return jnp.where(ad < 1.0, 0.5 * d * d, ad - 0.5)


def _body(reg_h, gt_h, hp_h, cls_h, kp_h, um_h, out_h,
          reg_v, gt_v, hp_v, cls_v, kp_v, um_v, outv):
    cid = lax.axis_index("c")
    sid = lax.axis_index("s")
    wid = sid * NC + cid

    lane = jnp.arange(L, dtype=jnp.int32)
    zf = jnp.zeros((L,), jnp.float32)
    c0 = jnp.zeros((L,), jnp.int32)
    c1 = jnp.full((L,), 1, jnp.int32)

    def grp_step(g, accs):
        a_cls, a_reg, a_kp, a_ncls, a_nreg, a_nkp = accs
        g16 = g * L
        li_l = g16 + lane

        li = jnp.full((L,), NP - 1, jnp.int32)
        anyb = jnp.zeros((L,), jnp.int32) > 0
        for t in range(NP):
            hb = hp_v[t, pl.ds(g16, L)] == 1
            li = jnp.where(hb, t, li)
            anyb = anyb | hb
        valid = (um_v[pl.ds(g16, L)] != 0) & anyb

        gx = plsc.load_gather(gt_v, [li, c0, li_l])
        gy = plsc.load_gather(gt_v, [li, c1, li_l])
        d2s = []
        for j in range(NM):
            pj = li + j * NP
            rx = plsc.load_gather(reg_v, [pj, c0, li_l])
            ry = plsc.load_gather(reg_v, [pj, c1, li_l])
            dx = rx - gx
            dy = ry - gy
            d2s.append(dx * dx + dy * dy)
        mind2 = d2s[0]
        minid = jnp.zeros((L,), jnp.int32)
        for j in range(1, NM):
            lt = d2s[j] < mind2
            mind2 = jnp.where(lt, d2s[j], mind2)
            minid = jnp.where(lt, j, minid)
        mind = _sqrtv(mind2)

        mask0 = (mind < CLS_TH) & valid
        cmin = plsc.load_gather(cls_v, [minid, li_l])
        for j in range(NM):
            cj = cls_v[j, pl.ds(g16, L)]
            mgn = cmin - cj
            dj = _sqrtv(d2s[j])
            mm = mask0 & ((dj - mind) > CLS_IGNORE) & (mgn < MGN)
            a_cls = a_cls + jnp.where(mm, MGN - mgn, zf)
            a_ncls = a_ncls + jnp.where(mm, 1.0, zf)

        minid30 = minid * NP
        for t in range(NP):
            pt = minid30 + t
            rx = plsc.load_gather(reg_v, [pt, c0, li_l])
            ry = plsc.load_gather(reg_v, [pt, c1, li_l])
            gx2 = gt_v[t, 0, pl.ds(g16, L)]
            gy2 = gt_v[t, 1, pl.ds(g16, L)]
            mk = (hp_v[t, pl.ds(g16, L)] == 1) & valid
            e = _sl1(rx - gx2) + _sl1(ry - gy2)
            a_reg = a_reg + jnp.where(mk, e, zf)
            a_nreg = a_nreg + jnp.where(mk, 1.0, zf)

        minid4 = minid * 4
        for s, ts in enumerate(KP_STEPS):
            qs = minid4 + s
            kx = plsc.load_gather(kp_v, [qs, c0, li_l])
            ky = plsc.load_gather(kp_v, [qs, c1, li_l])
            gxs = gt_v[ts, 0, pl.ds(g16, L)]
            gys = gt_v[ts, 1, pl.ds(g16, L)]
            mk = (hp_v[ts, pl.ds(g16, L)] == 1) & valid
            e = _sl1(kx - gxs) + _sl1(ky - gys)
            a_kp = a_kp + jnp.where(mk, e, zf)
            a_nkp = a_nkp + jnp.where(mk, 1.0, zf)

        return (a_cls, a_reg, a_kp, a_ncls, a_nreg, a_nkp)

    def blk_step(bi, accs):
        b = wid * BPW + bi
        pltpu.sync_copy(reg_h.at[:, b], reg_v)
        pltpu.sync_copy(gt_h.at[:, b], gt_v)
        pltpu.sync_copy(hp_h.at[:, b], hp_v)
        pltpu.sync_copy(cls_h.at[:, b], cls_v)
        pltpu.sync_copy(kp_h.at[:, b], kp_v)
        pltpu.sync_copy(um_h.at[b], um_v)
        return lax.fori_loop(0, GRP, grp_step, accs)

    accs = lax.fori_loop(0, BPW, blk_step, (zf, zf, zf, zf, zf, zf))
    for i in range(6):
        outv[i, :] = accs[i]
    pltpu.sync_copy(outv, out_h.at[wid])


@jax.jit
def _run(regW, gtW, hpW, clsW, kpW, umW):
    mesh = plsc.VectorSubcoreMesh(core_axis_name="c", subcore_axis_name="s",
                                  num_cores=NC, num_subcores=NS)
    f = pl.kernel(
        _body,
        out_type=jax.ShapeDtypeStruct((NW, 6, L), jnp.float32),
        mesh=mesh,
        scratch_types=[
            pltpu.VMEM((NM * NP, 2, B), jnp.float32),
            pltpu.VMEM((NP, 2, B), jnp.float32),
            pltpu.VMEM((NP, B), jnp.int32),
            pltpu.VMEM((NM, B), jnp.float32),
            pltpu.VMEM((NM * 4, 2, B), jnp.float32),
            pltpu.VMEM((B,), jnp.int32),
            pltpu.VMEM((6, L), jnp.float32),
        ],
        compiler_params=pltpu.CompilerParams(
            needs_layout_passes=False,
            use_tc_tiling_on_sc=False,
        ),
    )
    return f(regW, gtW, hpW, clsW, kpW, umW)


def kernel(cls, reg, key_points, gt_preds, has_preds, update_mask):
    regW = reg.reshape(B, B, NM, NP, 2).transpose(2, 3, 0, 4, 1) \
              .reshape(NM * NP, B, 2, B)
    gtW = gt_preds.reshape(B, B, NP, 2).transpose(2, 0, 3, 1) \
                  .reshape(NP, B, 2, B)
    hpW = has_preds.T.reshape(NP, B, B).astype(jnp.int32)
    clsW = cls.T.reshape(NM, B, B)
    kpW = key_points.reshape(B, B, NM, 4, 2).transpose(2, 3, 0, 4, 1) \
                    .reshape(NM * 4, B, 2, B)
    umW = update_mask.astype(jnp.int32).reshape(B, B)
    out = _run(regW, gtW, hpW, clsW, kpW, umW)
    s = out.sum(axis=(0, 2))
    return (s[0], s[1], s[2],
            s[3].astype(jnp.int32), s[4].astype(jnp.int32),
            s[5].astype(jnp.int32))


# speedup vs baseline: 66.6414x; 1.2388x over previous
import jax
import jax.numpy as jnp
from jax import lax
from jax.experimental import pallas as pl
from jax.experimental.pallas import tpu as pltpu
from jax.experimental.pallas import tpu_sc as plsc

N = 16384
NM = 6
NP = 30
NC = 2
NS = 16
L = 16
NW = NC * NS
B = 128
NB = N // B
BPW = NB // NW
GRP = B // L

CLS_TH = 2.0
CLS_IGNORE = 0.2
MGN = 0.2
KP_STEPS = (NP - 1, 9, 19, NP - 1)


def _sqrtv(x):
    i = plsc.bitcast(x, jnp.int32)
    i = 0x5F3759DF - lax.shift_right_logical(i, 1)
    y = plsc.bitcast(i, jnp.float32)
    for _ in range(3):
        y = y * (1.5 - 0.5 * x * y * y)
    return jnp.where(x > 0.0, x * y, 0.0)


def _sl1(d):
    ad = jnp.abs(d)
    return jnp.where(ad < 1.0, 0.5 * d * d, ad - 0.5)


def _body(reg_h, gt_h, hp_h, cls_h, kp_h, um_h, out_h,
          reg_v0, gt_v0, hp_v0, cls_v0,
          reg_v1, gt_v1, hp_v1, cls_v1,
          kp_v, um_v, outv, sem0, sem1):
    cid = lax.axis_index("c")
    sid = lax.axis_index("s")
    wid = sid * NC + cid

    lane = jnp.arange(L, dtype=jnp.int32)
    zf = jnp.zeros((L,), jnp.float32)
    c0 = jnp.zeros((L,), jnp.int32)
    c1 = jnp.full((L,), 1, jnp.int32)

    bufs = ((reg_v0, gt_v0, hp_v0, cls_v0),
            (reg_v1, gt_v1, hp_v1, cls_v1))
    sems = (sem0, sem1)

    def dma_block(b, k):
        reg_v, gt_v, hp_v, cls_v = bufs[k]
        sem = sems[k]
        return (
            pltpu.async_copy(reg_h.at[:, b], reg_v, sem),
            pltpu.async_copy(gt_h.at[:, b], gt_v, sem),
            pltpu.async_copy(hp_h.at[:, b], hp_v, sem),
            pltpu.async_copy(cls_h.at[:, b], cls_v, sem),
        )

    def make_grp_step(k):
        reg_v, gt_v, hp_v, cls_v = bufs[k]

        def grp_step(g, accs):
            a_cls, a_reg, a_kp, a_ncls, a_nreg, a_nkp = accs
            g16 = g * L
            li_l = g16 + lane

            vals = [jnp.where(hp_v[t, pl.ds(g16, L)] == 1, t, -1)
                    for t in range(NP)]
            while len(vals) > 1:
                nxt = [jnp.maximum(vals[i], vals[i + 1])
                       for i in range(0, len(vals) - 1, 2)]
                if len(vals) % 2:
                    nxt.append(vals[-1])
                vals = nxt
            anyb = vals[0] >= 0
            li = jnp.where(anyb, vals[0], NP - 1)
            valid = (um_v[pl.ds(g16, L)] != 0) & anyb

            gx = plsc.load_gather(gt_v, [li, c0, li_l])
            gy = plsc.load_gather(gt_v, [li, c1, li_l])
            d2s = []
            for j in range(NM):
                pj = li + j * NP
                rx = plsc.load_gather(reg_v, [pj, c0, li_l])
                ry = plsc.load_gather(reg_v, [pj, c1, li_l])
                dx = rx - gx
                dy = ry - gy
                d2s.append(dx * dx + dy * dy)
            mind2 = d2s[0]
            minid = jnp.zeros((L,), jnp.int32)
            for j in range(1, NM):
                lt = d2s[j] < mind2
                mind2 = jnp.where(lt, d2s[j], mind2)
                minid = jnp.where(lt, j, minid)
            mind = _sqrtv(mind2)

            mask0 = (mind < CLS_TH) & valid
            cmin = plsc.load_gather(cls_v, [minid, li_l])
            for j in range(NM):
                cj = cls_v[j, pl.ds(g16, L)]
                mgn = cmin - cj
                dj = _sqrtv(d2s[j])
                mm = mask0 & ((dj - mind) > CLS_IGNORE) & (mgn < MGN)
                a_cls = a_cls + jnp.where(mm, MGN - mgn, zf)
                a_ncls = a_ncls + jnp.where(mm, 1.0, zf)

            minid30 = minid * NP
            for t in range(NP):
                pt = minid30 + t
                rx = plsc.load_gather(reg_v, [pt, c0, li_l])
                ry = plsc.load_gather(reg_v, [pt, c1, li_l])
                gx2 = gt_v[t, 0, pl.ds(g16, L)]
                gy2 = gt_v[t, 1, pl.ds(g16, L)]
                mk = (hp_v[t, pl.ds(g16, L)] == 1) & valid
                e = _sl1(rx - gx2) + _sl1(ry - gy2)
                a_reg = a_reg + jnp.where(mk, e, zf)
                a_nreg = a_nreg + jnp.where(mk, 1.0, zf)

            minid4 = minid * 4
            for s, ts in enumerate(KP_STEPS):
                qs = minid4 + s
                kx = plsc.load_gather(kp_v, [qs, c0, li_l])
                ky = plsc.load_gather(kp_v, [qs, c1, li_l])
                gxs = gt_v[ts, 0, pl.ds(g16, L)]
                gys = gt_v[ts, 1, pl.ds(g16, L)]
                mk = (hp_v[ts, pl.ds(g16, L)] == 1) & valid
                e = _sl1(kx - gxs) + _sl1(ky - gys)
                a_kp = a_kp + jnp.where(mk, e, zf)
                a_nkp = a_nkp + jnp.where(mk, 1.0, zf)

            return (a_cls, a_reg, a_kp, a_ncls, a_nreg, a_nkp)

        return grp_step

    b0 = wid * BPW
    accs = (zf, zf, zf, zf, zf, zf)
    pend = dma_block(b0, 0)
    for bi in range(BPW):
        k = bi % 2
        for d in pend:
            d.wait()
        if bi + 1 < BPW:
            pend = dma_block(b0 + bi + 1, 1 - k)
        pltpu.sync_copy(kp_h.at[:, b0 + bi], kp_v)
        pltpu.sync_copy(um_h.at[b0 + bi], um_v)
        accs = lax.fori_loop(0, GRP, make_grp_step(k), accs)

    for i in range(6):
        outv[i, :] = accs[i]
    pltpu.sync_copy(outv, out_h.at[wid])


@jax.jit
def _run(regW, gtW, hpW, clsW, kpW, umW):
    mesh = plsc.VectorSubcoreMesh(core_axis_name="c", subcore_axis_name="s",
                                  num_cores=NC, num_subcores=NS)
    f = pl.kernel(
        _body,
        out_type=jax.ShapeDtypeStruct((NW, 6, L), jnp.float32),
        mesh=mesh,
        scratch_types=(
            [
                pltpu.VMEM((NM * NP, 2, B), jnp.float32),
                pltpu.VMEM((NP, 2, B), jnp.float32),
                pltpu.VMEM((NP, B), jnp.int32),
                pltpu.VMEM((NM, B), jnp.float32),
            ] * 2
            + [
                pltpu.VMEM((NM * 4, 2, B), jnp.float32),
                pltpu.VMEM((B,), jnp.int32),
                pltpu.VMEM((6, L), jnp.float32),
                pltpu.SemaphoreType.DMA,
                pltpu.SemaphoreType.DMA,
            ]
        ),
        compiler_params=pltpu.CompilerParams(
            needs_layout_passes=False,
            use_tc_tiling_on_sc=False,
        ),
    )
    return f(regW, gtW, hpW, clsW, kpW, umW)


def kernel(cls, reg, key_points, gt_preds, has_preds, update_mask):
    regW = reg.reshape(B, B, NM, NP, 2).transpose(2, 3, 0, 4, 1) \
              .reshape(NM * NP, B, 2, B)
    gtW = gt_preds.reshape(B, B, NP, 2).transpose(2, 0, 3, 1) \
                  .reshape(NP, B, 2, B)
    hpW = has_preds.T.reshape(NP, B, B).astype(jnp.int32)
    clsW = cls.T.reshape(NM, B, B)
    kpW = key_points.reshape(B, B, NM, 4, 2).transpose(2, 3, 0, 4, 1) \
                    .reshape(NM * 4, B, 2, B)
    umW = update_mask.astype(jnp.int32).reshape(B, B)
    out = _run(regW, gtW, hpW, clsW, kpW, umW)
    s = out.sum(axis=(0, 2))
    return (s[0], s[1], s[2],
            s[3].astype(jnp.int32), s[4].astype(jnp.int32),
            s[5].astype(jnp.int32))
